# baseline (speedup 1.0000x reference)
import jax
import jax.numpy as jnp
from jax import lax
from jax.experimental import pallas as pl
from jax.experimental.pallas import tpu as pltpu
from jax.experimental.pallas import tpu_sc as plsc

_N = 10000
_E = 320000
_H = 128
_NC = 2
_NS = 16
_NW = _NC * _NS
_EPT = _E // _NW
_CH = 40
_NCHUNK = _EPT // _CH
_BCH = 50
_NB = _NCHUNK // _BCH
_RPT = 624
_TAIL = _N - _NS * _RPT
_ZN = _RPT // _CH
_ZREM = _RPT - _ZN * _CH

_mesh = plsc.VectorSubcoreMesh(core_axis_name="c", subcore_axis_name="s")


def _fill(ref, nrows, ncols, value):
    v = jnp.full((16,), value, jnp.float32)
    for r in range(nrows):
        for j in range(ncols // 16):
            ref[r, pl.ds(j * 16, 16)] = v


def _clear_rows(zb, acc, s):
    base = s * _RPT
    for t in range(_ZN):
        pltpu.sync_copy(zb, acc.at[pl.ds(base + t * _CH, _CH)])
    if _ZREM:
        pltpu.sync_copy(zb.at[pl.ds(0, _ZREM)],
                        acc.at[pl.ds(base + _ZN * _CH, _ZREM)])

    @pl.when(s == 0)
    def _():
        pltpu.sync_copy(zb.at[pl.ds(0, _TAIL)],
                        acc.at[pl.ds(_NS * _RPT, _TAIL)])


def _copy_out(acc, out_hbm, c, s):
    row0 = s * _RPT
    pltpu.sync_copy(acc.at[pl.ds(row0, _RPT)],
                    out_hbm.at[pl.ds(c * _N + row0, _RPT)])

    @pl.when(s == 0)
    def _():
        pltpu.sync_copy(acc.at[pl.ds(_NS * _RPT, _TAIL)],
                        out_hbm.at[pl.ds(c * _N + _NS * _RPT, _TAIL)])


def _agg_body(u_hbm, src4_hbm, dst4_hbm, out_hbm, acc, sb0, db0, sb1, db1,
              rows0, rows1, semi0, semi1, sem0, sem1, ssem0, ssem1):
    c = lax.axis_index("c")
    s = lax.axis_index("s")
    wid = c * _NS + s
    sb = (sb0, sb1)
    db = (db0, db1)
    semi = (semi0, semi1)
    rows = (rows0, rows1)
    sems = (sem0, sem1)
    ssems = (ssem0, ssem1)

    def idx_issue(blk):
        sl = blk % 2
        pltpu.async_copy(src4_hbm.at[wid, blk], sb[sl], semi[sl])
        pltpu.async_copy(dst4_hbm.at[wid, blk], db[sl], semi[sl])

    def idx_wait(sl):
        pltpu.make_async_copy(src4_hbm.at[wid, 0], sb[sl], semi[sl]).wait()
        pltpu.make_async_copy(dst4_hbm.at[wid, 0], db[sl], semi[sl]).wait()

    idx_issue(0)
    idx_issue(1)
    _fill(rows0, _CH, _H, 0.0)
    _clear_rows(rows0, acc, s)
    idx_wait(0)
    plsc.subcore_barrier()

    def issue(sl, q, b):
        pltpu.async_copy(u_hbm.at[sb[sl].at[q]], rows[b], sems[b])

    def consume(sl, q, b):
        pltpu.make_async_copy(u_hbm.at[pl.ds(0, _CH)], rows[b],
                              sems[b]).wait()
        pltpu.async_copy(rows[b], acc.at[db[sl].at[q]], ssems[b], add=True)

    def drain_scat(b):
        pltpu.make_async_copy(u_hbm.at[pl.ds(0, _CH)], rows[b],
                              ssems[b]).wait()

    issue(0, 0, 0)
    issue(0, 1, 1)

    for blk in range(_NB):
        sl = blk % 2
        nsl = 1 - sl
        last = blk == _NB - 1

        @pl.loop(0, _BCH - 2, step=2)
        def _(q, sl=sl):
            consume(sl, q, 0)
            consume(sl, q + 1, 1)
            drain_scat(0)
            issue(sl, q + 2, 0)
            drain_scat(1)
            issue(sl, q + 3, 1)

        if not last:
            idx_wait(nsl)
        consume(sl, _BCH - 2, 0)
        consume(sl, _BCH - 1, 1)
        drain_scat(0)
        drain_scat(1)
        if not last:
            issue(nsl, 0, 0)
            issue(nsl, 1, 1)
            if blk + 2 < _NB:
                idx_issue(blk + 2)

    plsc.subcore_barrier()
    _copy_out(acc, out_hbm, c, s)


_sc_agg = pl.kernel(
    _agg_body,
    out_type=jax.ShapeDtypeStruct((2 * _N, _H), jnp.float32),
    mesh=_mesh,
    scratch_types=[
        pltpu.VMEM_SHARED((_N, _H), jnp.float32),
        pltpu.VMEM((_BCH, _CH), jnp.int32),
        pltpu.VMEM((_BCH, _CH), jnp.int32),
        pltpu.VMEM((_BCH, _CH), jnp.int32),
        pltpu.VMEM((_BCH, _CH), jnp.int32),
        pltpu.VMEM((_CH, _H), jnp.float32),
        pltpu.VMEM((_CH, _H), jnp.float32),
        pltpu.SemaphoreType.DMA,
        pltpu.SemaphoreType.DMA,
        pltpu.SemaphoreType.DMA,
        pltpu.SemaphoreType.DMA,
        pltpu.SemaphoreType.DMA,
        pltpu.SemaphoreType.DMA,
    ],
)

_CNTW = 128


def _cnt_body(dst3_hbm, out_hbm, acc, dst_v, ones, zb, semi):
    c = lax.axis_index("c")
    s = lax.axis_index("s")
    wid = c * _NS + s
    p = pltpu.async_copy(dst3_hbm.at[wid], dst_v, semi)
    _fill(zb, _CH, _CNTW, 0.0)
    _clear_rows(zb, acc, s)
    _fill(ones, _CH, _CNTW, 1.0)
    p.wait()
    plsc.subcore_barrier()

    @pl.loop(0, _NCHUNK)
    def _(q):
        pltpu.sync_copy(ones, acc.at[dst_v.at[q]], add=True)

    plsc.subcore_barrier()
    _copy_out(acc, out_hbm, c, s)


_sc_cnt = pl.kernel(
    _cnt_body,
    out_type=jax.ShapeDtypeStruct((2 * _N, _CNTW), jnp.float32),
    mesh=_mesh,
    scratch_types=[
        pltpu.VMEM_SHARED((_N, _CNTW), jnp.float32),
        pltpu.VMEM((_NCHUNK, _CH), jnp.int32),
        pltpu.VMEM((_CH, _CNTW), jnp.float32),
        pltpu.VMEM((_CH, _CNTW), jnp.float32),
        pltpu.SemaphoreType.DMA,
    ],
)

_BLK = 1000
_G = _N // _BLK

_full_w = pl.BlockSpec((_H, _H), lambda i: (0, 0))
_full_b = pl.BlockSpec((1, _H), lambda i: (0, 0))
_row_blk = pl.BlockSpec((_BLK, _H), lambda i: (i, 0))
_row_blk_hi = pl.BlockSpec((_BLK, _H), lambda i: (i + _G, 0))
_cnt_blk = pl.BlockSpec((_BLK, _CNTW), lambda i: (i, 0))
_cnt_blk_hi = pl.BlockSpec((_BLK, _CNTW), lambda i: (i + _G, 0))


def _enc_body(x_r, c0_r, c1_r, w1_r, b1_r, w2_r, b2_r, w0_r, u_r, dv_r):
    deg = c0_r[:, 0:1] + c1_r[:, 0:1] + 1.0
    dinv = lax.rsqrt(deg)
    h = jnp.maximum(
        jnp.dot(x_r[...], w1_r[...], preferred_element_type=jnp.float32)
        + b1_r[...], 0.0)
    h = jnp.maximum(
        jnp.dot(h, w2_r[...], preferred_element_type=jnp.float32)
        + b2_r[...], 0.0)
    u_r[...] = dinv * jnp.dot(h, w0_r[...], preferred_element_type=jnp.float32)
    dv_r[...] = jnp.broadcast_to(dinv, (_BLK, _H))


_enc_call = pl.pallas_call(
    _enc_body,
    grid=(_G,),
    in_specs=[_row_blk, _cnt_blk, _cnt_blk_hi, _full_w, _full_b, _full_w,
              _full_b, _full_w],
    out_specs=[_row_blk, _row_blk],
    out_shape=[jax.ShapeDtypeStruct((_N, _H), jnp.float32),
               jax.ShapeDtypeStruct((_N, _H), jnp.float32)],
)


def _mid_body(s0_r, s1_r, u_r, dv_r, b_r, w_r, o_r):
    dinv = dv_r[...]
    h = jnp.maximum(dinv * (s0_r[...] + s1_r[...] + u_r[...]) + b_r[...], 0.0)
    o_r[...] = dinv * jnp.dot(h, w_r[...], preferred_element_type=jnp.float32)


_mid_call = pl.pallas_call(
    _mid_body,
    grid=(_G,),
    in_specs=[_row_blk, _row_blk_hi, _row_blk, _row_blk, _full_b, _full_w],
    out_specs=_row_blk,
    out_shape=jax.ShapeDtypeStruct((_N, _H), jnp.float32),
)


def _dec_body(s0_r, s1_r, u_r, dv_r, b_r, w1_r, b1_r, w2_r, b2_r, o_r):
    h9 = dv_r[...] * (s0_r[...] + s1_r[...] + u_r[...]) + b_r[...]
    d = jnp.maximum(
        jnp.dot(h9, w1_r[...], preferred_element_type=jnp.float32)
        + b1_r[...], 0.0)
    o_r[...] = jnp.dot(d, w2_r[...], preferred_element_type=jnp.float32) \
        + b2_r[...]


_dec_call = pl.pallas_call(
    _dec_body,
    grid=(_G,),
    in_specs=[_row_blk, _row_blk_hi, _row_blk, _row_blk, _full_b, _full_w,
              _full_b, pl.BlockSpec((_H, 3), lambda i: (0, 0)),
              pl.BlockSpec((1, 3), lambda i: (0, 0))],
    out_specs=pl.BlockSpec((_BLK, 3), lambda i: (i, 0)),
    out_shape=jax.ShapeDtypeStruct((_N, 3), jnp.float32),
)


def kernel(x, edge_index, enc_W1, enc_b1, enc_W2, enc_b2, gcn_W, gcn_b,
           dec_W1, dec_b1, dec_W2, dec_b2):
    src = edge_index[0].reshape(_NW, _NB, _BCH, _CH)
    dst = edge_index[1].reshape(_NW, _NB, _BCH, _CH)
    cnt = _sc_cnt(edge_index[1].reshape(_NW, _NCHUNK, _CH))
    u, dinvB = _enc_call(x, cnt, cnt, enc_W1, enc_b1.reshape(1, _H),
                         enc_W2, enc_b2.reshape(1, _H), gcn_W[0])
    L = gcn_W.shape[0]
    for i in range(L):
        S = _sc_agg(u, src, dst)
        b_i = gcn_b[i].reshape(1, _H)
        if i < L - 1:
            u = _mid_call(S, S, u, dinvB, b_i, gcn_W[i + 1])
        else:
            out = _dec_call(S, S, u, dinvB, b_i, dec_W1,
                            dec_b1.reshape(1, _H), dec_W2,
                            dec_b2.reshape(1, 3))
    return out


# speedup vs baseline: 1.1338x; 1.1338x over previous
import jax
import jax.numpy as jnp
from jax import lax
from jax.experimental import pallas as pl
from jax.experimental.pallas import tpu as pltpu
from jax.experimental.pallas import tpu_sc as plsc

_N = 10000
_E = 320000
_H = 128
_NC = 2
_NS = 16
_NW = _NC * _NS
_EPT = _E // _NW
_CH = 40
_NCHUNK = _EPT // _CH
_BCH = 50
_NB = _NCHUNK // _BCH
_RPT = 624
_TAIL = _N - _NS * _RPT
_ZN = _RPT // _CH
_ZREM = _RPT - _ZN * _CH

_mesh = plsc.VectorSubcoreMesh(core_axis_name="c", subcore_axis_name="s")


def _fill(ref, nrows, ncols, value):
    v = jnp.full((16,), value, jnp.float32)
    for r in range(nrows):
        for j in range(ncols // 16):
            ref[r, pl.ds(j * 16, 16)] = v


def _clear_rows(zb, acc, s):
    base = s * _RPT
    for t in range(_ZN):
        pltpu.sync_copy(zb, acc.at[pl.ds(base + t * _CH, _CH)])
    if _ZREM:
        pltpu.sync_copy(zb.at[pl.ds(0, _ZREM)],
                        acc.at[pl.ds(base + _ZN * _CH, _ZREM)])

    @pl.when(s == 0)
    def _():
        pltpu.sync_copy(zb.at[pl.ds(0, _TAIL)],
                        acc.at[pl.ds(_NS * _RPT, _TAIL)])


def _copy_out(acc, out_hbm, c, s):
    row0 = s * _RPT
    pltpu.sync_copy(acc.at[pl.ds(row0, _RPT)],
                    out_hbm.at[pl.ds(c * _N + row0, _RPT)])

    @pl.when(s == 0)
    def _():
        pltpu.sync_copy(acc.at[pl.ds(_NS * _RPT, _TAIL)],
                        out_hbm.at[pl.ds(c * _N + _NS * _RPT, _TAIL)])


def _agg_body(u_hbm, src4_hbm, dst4_hbm, out_hbm, acc, sb0, db0, sb1, db1,
              rows0, rows1, semi0, semi1, sem0, sem1, ssem0, ssem1):
    c = lax.axis_index("c")
    s = lax.axis_index("s")
    wid = c * _NS + s
    sb = (sb0, sb1)
    db = (db0, db1)
    semi = (semi0, semi1)
    rows = (rows0, rows1)
    sems = (sem0, sem1)
    ssems = (ssem0, ssem1)

    def idx_issue(blk):
        sl = blk % 2
        pltpu.async_copy(src4_hbm.at[wid, blk], sb[sl], semi[sl])
        pltpu.async_copy(dst4_hbm.at[wid, blk], db[sl], semi[sl])

    def idx_wait(sl):
        pltpu.make_async_copy(src4_hbm.at[wid, 0], sb[sl], semi[sl]).wait()
        pltpu.make_async_copy(dst4_hbm.at[wid, 0], db[sl], semi[sl]).wait()

    idx_issue(0)
    idx_issue(1)
    _fill(rows0, _CH, _H, 0.0)
    _clear_rows(rows0, acc, s)
    idx_wait(0)
    plsc.subcore_barrier()

    def issue(sl, q, b):
        pltpu.async_copy(u_hbm.at[sb[sl].at[q]], rows[b], sems[b])

    def consume(sl, q, b):
        pltpu.make_async_copy(u_hbm.at[pl.ds(0, _CH)], rows[b],
                              sems[b]).wait()
        pltpu.sync_copy(rows[b], acc.at[db[sl].at[q]], add=True)

    issue(0, 0, 0)
    issue(0, 1, 1)

    for blk in range(_NB):
        sl = blk % 2
        nsl = 1 - sl
        last = blk == _NB - 1

        @pl.loop(0, _BCH - 2, step=2)
        def _(q, sl=sl):
            consume(sl, q, 0)
            issue(sl, q + 2, 0)
            consume(sl, q + 1, 1)
            issue(sl, q + 3, 1)

        if not last:
            idx_wait(nsl)
        consume(sl, _BCH - 2, 0)
        if not last:
            issue(nsl, 0, 0)
        consume(sl, _BCH - 1, 1)
        if not last:
            issue(nsl, 1, 1)
            if blk + 2 < _NB:
                idx_issue(blk + 2)

    plsc.subcore_barrier()
    _copy_out(acc, out_hbm, c, s)


_sc_agg = pl.kernel(
    _agg_body,
    out_type=jax.ShapeDtypeStruct((2 * _N, _H), jnp.float32),
    mesh=_mesh,
    scratch_types=[
        pltpu.VMEM_SHARED((_N, _H), jnp.float32),
        pltpu.VMEM((_BCH, _CH), jnp.int32),
        pltpu.VMEM((_BCH, _CH), jnp.int32),
        pltpu.VMEM((_BCH, _CH), jnp.int32),
        pltpu.VMEM((_BCH, _CH), jnp.int32),
        pltpu.VMEM((_CH, _H), jnp.float32),
        pltpu.VMEM((_CH, _H), jnp.float32),
        pltpu.SemaphoreType.DMA,
        pltpu.SemaphoreType.DMA,
        pltpu.SemaphoreType.DMA,
        pltpu.SemaphoreType.DMA,
        pltpu.SemaphoreType.DMA,
        pltpu.SemaphoreType.DMA,
    ],
)

_CNTW = 128


def _cnt_body(dst3_hbm, out_hbm, acc, dst_v, ones, zb, semi):
    c = lax.axis_index("c")
    s = lax.axis_index("s")
    wid = c * _NS + s
    p = pltpu.async_copy(dst3_hbm.at[wid], dst_v, semi)
    _fill(zb, _CH, _CNTW, 0.0)
    _clear_rows(zb, acc, s)
    _fill(ones, _CH, _CNTW, 1.0)
    p.wait()
    plsc.subcore_barrier()

    @pl.loop(0, _NCHUNK)
    def _(q):
        pltpu.sync_copy(ones, acc.at[dst_v.at[q]], add=True)

    plsc.subcore_barrier()
    _copy_out(acc, out_hbm, c, s)


_sc_cnt = pl.kernel(
    _cnt_body,
    out_type=jax.ShapeDtypeStruct((2 * _N, _CNTW), jnp.float32),
    mesh=_mesh,
    scratch_types=[
        pltpu.VMEM_SHARED((_N, _CNTW), jnp.float32),
        pltpu.VMEM((_NCHUNK, _CH), jnp.int32),
        pltpu.VMEM((_CH, _CNTW), jnp.float32),
        pltpu.VMEM((_CH, _CNTW), jnp.float32),
        pltpu.SemaphoreType.DMA,
    ],
)

_BLK = 1000
_G = _N // _BLK

_full_w = pl.BlockSpec((_H, _H), lambda i: (0, 0))
_full_b = pl.BlockSpec((1, _H), lambda i: (0, 0))
_row_blk = pl.BlockSpec((_BLK, _H), lambda i: (i, 0))
_row_blk_hi = pl.BlockSpec((_BLK, _H), lambda i: (i + _G, 0))
_cnt_blk = pl.BlockSpec((_BLK, _CNTW), lambda i: (i, 0))
_cnt_blk_hi = pl.BlockSpec((_BLK, _CNTW), lambda i: (i + _G, 0))


def _enc1_body(x_r, w1_r, b1_r, w2_r, b2_r, h_r):
    h = jnp.maximum(
        jnp.dot(x_r[...], w1_r[...], preferred_element_type=jnp.float32)
        + b1_r[...], 0.0)
    h_r[...] = jnp.maximum(
        jnp.dot(h, w2_r[...], preferred_element_type=jnp.float32)
        + b2_r[...], 0.0)


_enc1_call = pl.pallas_call(
    _enc1_body,
    grid=(_G,),
    in_specs=[_row_blk, _full_w, _full_b, _full_w, _full_b],
    out_specs=_row_blk,
    out_shape=jax.ShapeDtypeStruct((_N, _H), jnp.float32),
)


def _enc2_body(h_r, c0_r, c1_r, w0_r, u_r, dv_r):
    deg = c0_r[:, 0:1] + c1_r[:, 0:1] + 1.0
    dinv = lax.rsqrt(deg)
    u_r[...] = dinv * jnp.dot(h_r[...], w0_r[...],
                              preferred_element_type=jnp.float32)
    dv_r[...] = jnp.broadcast_to(dinv, (_BLK, _H))


_enc2_call = pl.pallas_call(
    _enc2_body,
    grid=(_G,),
    in_specs=[_row_blk, _cnt_blk, _cnt_blk_hi, _full_w],
    out_specs=[_row_blk, _row_blk],
    out_shape=[jax.ShapeDtypeStruct((_N, _H), jnp.float32),
               jax.ShapeDtypeStruct((_N, _H), jnp.float32)],
)


def _mid_body(s0_r, s1_r, u_r, dv_r, b_r, w_r, o_r):
    dinv = dv_r[...]
    h = jnp.maximum(dinv * (s0_r[...] + s1_r[...] + u_r[...]) + b_r[...], 0.0)
    o_r[...] = dinv * jnp.dot(h, w_r[...], preferred_element_type=jnp.float32)


_mid_call = pl.pallas_call(
    _mid_body,
    grid=(_G,),
    in_specs=[_row_blk, _row_blk_hi, _row_blk, _row_blk, _full_b, _full_w],
    out_specs=_row_blk,
    out_shape=jax.ShapeDtypeStruct((_N, _H), jnp.float32),
)


def _dec_body(s0_r, s1_r, u_r, dv_r, b_r, w1_r, b1_r, w2_r, b2_r, o_r):
    h9 = dv_r[...] * (s0_r[...] + s1_r[...] + u_r[...]) + b_r[...]
    d = jnp.maximum(
        jnp.dot(h9, w1_r[...], preferred_element_type=jnp.float32)
        + b1_r[...], 0.0)
    o_r[...] = jnp.dot(d, w2_r[...], preferred_element_type=jnp.float32) \
        + b2_r[...]


_dec_call = pl.pallas_call(
    _dec_body,
    grid=(_G,),
    in_specs=[_row_blk, _row_blk_hi, _row_blk, _row_blk, _full_b, _full_w,
              _full_b, pl.BlockSpec((_H, 3), lambda i: (0, 0)),
              pl.BlockSpec((1, 3), lambda i: (0, 0))],
    out_specs=pl.BlockSpec((_BLK, 3), lambda i: (i, 0)),
    out_shape=jax.ShapeDtypeStruct((_N, 3), jnp.float32),
)


def kernel(x, edge_index, enc_W1, enc_b1, enc_W2, enc_b2, gcn_W, gcn_b,
           dec_W1, dec_b1, dec_W2, dec_b2):
    src = edge_index[0].reshape(_NW, _NB, _BCH, _CH)
    dst = edge_index[1].reshape(_NW, _NB, _BCH, _CH)
    cnt = _sc_cnt(edge_index[1].reshape(_NW, _NCHUNK, _CH))
    h2 = _enc1_call(x, enc_W1, enc_b1.reshape(1, _H),
                    enc_W2, enc_b2.reshape(1, _H))
    u, dinvB = _enc2_call(h2, cnt, cnt, gcn_W[0])
    L = gcn_W.shape[0]
    for i in range(L):
        S = _sc_agg(u, src, dst)
        b_i = gcn_b[i].reshape(1, _H)
        if i < L - 1:
            u = _mid_call(S, S, u, dinvB, b_i, gcn_W[i + 1])
        else:
            out = _dec_call(S, S, u, dinvB, b_i, dec_W1,
                            dec_b1.reshape(1, _H), dec_W2,
                            dec_b2.reshape(1, 3))
    return out


# speedup vs baseline: 1.4362x; 1.2667x over previous
import jax
import jax.numpy as jnp
from jax import lax
from jax.experimental import pallas as pl
from jax.experimental.pallas import tpu as pltpu
from jax.experimental.pallas import tpu_sc as plsc

_N = 10000
_E = 320000
_H = 128
_NC = 2
_NS = 16
_NW = _NC * _NS
_EPT = _E // _NW
_CH = 40
_NCHUNK = _EPT // _CH
_BCH = 25
_NB = _NCHUNK // _BCH
_RPT = 624
_TAIL = _N - _NS * _RPT
_ZN = _RPT // _CH
_ZREM = _RPT - _ZN * _CH

_mesh = plsc.VectorSubcoreMesh(core_axis_name="c", subcore_axis_name="s")


def _fill(ref, nrows, ncols, value):
    v = jnp.full((16,), value, jnp.float32)
    for r in range(nrows):
        for j in range(ncols // 16):
            ref[r, pl.ds(j * 16, 16)] = v


def _clear_rows(zb, acc, s):
    base = s * _RPT
    for t in range(_ZN):
        pltpu.sync_copy(zb, acc.at[pl.ds(base + t * _CH, _CH)])
    if _ZREM:
        pltpu.sync_copy(zb.at[pl.ds(0, _ZREM)],
                        acc.at[pl.ds(base + _ZN * _CH, _ZREM)])

    @pl.when(s == 0)
    def _():
        pltpu.sync_copy(zb.at[pl.ds(0, _TAIL)],
                        acc.at[pl.ds(_NS * _RPT, _TAIL)])


def _copy_out(acc, out_hbm, c, s):
    row0 = s * _RPT
    pltpu.sync_copy(acc.at[pl.ds(row0, _RPT)],
                    out_hbm.at[pl.ds(c * _N + row0, _RPT)])

    @pl.when(s == 0)
    def _():
        pltpu.sync_copy(acc.at[pl.ds(_NS * _RPT, _TAIL)],
                        out_hbm.at[pl.ds(c * _N + _NS * _RPT, _TAIL)])


def _agg_body(u_hbm, src4_hbm, dst4_hbm, out_hbm, acc, sb0, db0, sb1, db1,
              rows0, rows1, rows2, semi0, semi1, sem0, sem1, sem2):
    c = lax.axis_index("c")
    s = lax.axis_index("s")
    wid = c * _NS + s
    sb = (sb0, sb1)
    db = (db0, db1)
    semi = (semi0, semi1)
    rows = (rows0, rows1, rows2)
    sems = (sem0, sem1, sem2)

    def idx_issue(blk):
        sl = blk % 2
        pltpu.async_copy(src4_hbm.at[wid, blk], sb[sl], semi[sl])
        pltpu.async_copy(dst4_hbm.at[wid, blk], db[sl], semi[sl])

    def idx_wait(sl):
        pltpu.make_async_copy(src4_hbm.at[wid, 0], sb[sl], semi[sl]).wait()
        pltpu.make_async_copy(dst4_hbm.at[wid, 0], db[sl], semi[sl]).wait()

    idx_issue(0)
    idx_issue(1)
    _fill(rows0, _CH, _H, 0.0)
    _clear_rows(rows0, acc, s)
    idx_wait(0)
    plsc.subcore_barrier()

    def issue(sl, q, b):
        pltpu.async_copy(u_hbm.at[sb[sl].at[q]], rows[b], sems[b])

    def consume(sl, q, b):
        pltpu.make_async_copy(u_hbm.at[pl.ds(0, _CH)], rows[b],
                              sems[b]).wait()
        pltpu.sync_copy(rows[b], acc.at[db[sl].at[q]], add=True)

    issue(0, 0, 0)
    issue(0, 1, 1)
    issue(0, 2, 2)

    for blk in range(_NB):
        sl = blk % 2
        nsl = 1 - sl
        ph = blk % 3
        last = blk == _NB - 1

        @pl.loop(0, _BCH - 4, step=3)
        def _(q, sl=sl, ph=ph):
            consume(sl, q, ph)
            issue(sl, q + 3, ph)
            consume(sl, q + 1, (ph + 1) % 3)
            issue(sl, q + 4, (ph + 1) % 3)
            consume(sl, q + 2, (ph + 2) % 3)
            issue(sl, q + 5, (ph + 2) % 3)

        if not last:
            idx_wait(nsl)
        consume(sl, _BCH - 4, ph)
        issue(sl, _BCH - 1, ph)
        consume(sl, _BCH - 3, (ph + 1) % 3)
        if not last:
            issue(nsl, 0, (ph + 1) % 3)
        consume(sl, _BCH - 2, (ph + 2) % 3)
        if not last:
            issue(nsl, 1, (ph + 2) % 3)
        consume(sl, _BCH - 1, ph)
        if not last:
            issue(nsl, 2, ph)
            if blk + 2 < _NB:
                idx_issue(blk + 2)

    plsc.subcore_barrier()
    _copy_out(acc, out_hbm, c, s)


_sc_agg = pl.kernel(
    _agg_body,
    out_type=jax.ShapeDtypeStruct((2 * _N, _H), jnp.float32),
    mesh=_mesh,
    scratch_types=[
        pltpu.VMEM_SHARED((_N, _H), jnp.float32),
        pltpu.VMEM((_BCH, _CH), jnp.int32),
        pltpu.VMEM((_BCH, _CH), jnp.int32),
        pltpu.VMEM((_BCH, _CH), jnp.int32),
        pltpu.VMEM((_BCH, _CH), jnp.int32),
        pltpu.VMEM((_CH, _H), jnp.float32),
        pltpu.VMEM((_CH, _H), jnp.float32),
        pltpu.VMEM((_CH, _H), jnp.float32),
        pltpu.SemaphoreType.DMA,
        pltpu.SemaphoreType.DMA,
        pltpu.SemaphoreType.DMA,
        pltpu.SemaphoreType.DMA,
        pltpu.SemaphoreType.DMA,
    ],
)

_CNTW = 128


def _cnt_body(dst3_hbm, out_hbm, acc, dst_v, ones, zb, semi):
    c = lax.axis_index("c")
    s = lax.axis_index("s")
    wid = c * _NS + s
    p = pltpu.async_copy(dst3_hbm.at[wid], dst_v, semi)
    _fill(zb, _CH, _CNTW, 0.0)
    _clear_rows(zb, acc, s)
    _fill(ones, _CH, _CNTW, 1.0)
    p.wait()
    plsc.subcore_barrier()

    @pl.loop(0, _NCHUNK)
    def _(q):
        pltpu.sync_copy(ones, acc.at[dst_v.at[q]], add=True)

    plsc.subcore_barrier()
    _copy_out(acc, out_hbm, c, s)


_sc_cnt = pl.kernel(
    _cnt_body,
    out_type=jax.ShapeDtypeStruct((2 * _N, _CNTW), jnp.float32),
    mesh=_mesh,
    scratch_types=[
        pltpu.VMEM_SHARED((_N, _CNTW), jnp.float32),
        pltpu.VMEM((_NCHUNK, _CH), jnp.int32),
        pltpu.VMEM((_CH, _CNTW), jnp.float32),
        pltpu.VMEM((_CH, _CNTW), jnp.float32),
        pltpu.SemaphoreType.DMA,
    ],
)

_BLK = 1000
_G = _N // _BLK

_full_w = pl.BlockSpec((_H, _H), lambda i: (0, 0))
_full_b = pl.BlockSpec((1, _H), lambda i: (0, 0))
_row_blk = pl.BlockSpec((_BLK, _H), lambda i: (i, 0))
_row_blk_hi = pl.BlockSpec((_BLK, _H), lambda i: (i + _G, 0))
_cnt_blk = pl.BlockSpec((_BLK, _CNTW), lambda i: (i, 0))
_cnt_blk_hi = pl.BlockSpec((_BLK, _CNTW), lambda i: (i + _G, 0))


def _enc1_body(x_r, w1_r, b1_r, w2_r, b2_r, h_r):
    h = jnp.maximum(
        jnp.dot(x_r[...], w1_r[...], preferred_element_type=jnp.float32)
        + b1_r[...], 0.0)
    h_r[...] = jnp.maximum(
        jnp.dot(h, w2_r[...], preferred_element_type=jnp.float32)
        + b2_r[...], 0.0)


_enc1_call = pl.pallas_call(
    _enc1_body,
    grid=(_G,),
    in_specs=[_row_blk, _full_w, _full_b, _full_w, _full_b],
    out_specs=_row_blk,
    out_shape=jax.ShapeDtypeStruct((_N, _H), jnp.float32),
)


def _enc2_body(h_r, c0_r, c1_r, w0_r, u_r, dv_r):
    deg = c0_r[:, 0:1] + c1_r[:, 0:1] + 1.0
    dinv = lax.rsqrt(deg)
    u_r[...] = dinv * jnp.dot(h_r[...], w0_r[...],
                              preferred_element_type=jnp.float32)
    dv_r[...] = jnp.broadcast_to(dinv, (_BLK, _H))


_enc2_call = pl.pallas_call(
    _enc2_body,
    grid=(_G,),
    in_specs=[_row_blk, _cnt_blk, _cnt_blk_hi, _full_w],
    out_specs=[_row_blk, _row_blk],
    out_shape=[jax.ShapeDtypeStruct((_N, _H), jnp.float32),
               jax.ShapeDtypeStruct((_N, _H), jnp.float32)],
)


def _mid_body(s0_r, s1_r, u_r, dv_r, b_r, w_r, o_r):
    dinv = dv_r[...]
    h = jnp.maximum(dinv * (s0_r[...] + s1_r[...] + u_r[...]) + b_r[...], 0.0)
    o_r[...] = dinv * jnp.dot(h, w_r[...], preferred_element_type=jnp.float32)


_mid_call = pl.pallas_call(
    _mid_body,
    grid=(_G,),
    in_specs=[_row_blk, _row_blk_hi, _row_blk, _row_blk, _full_b, _full_w],
    out_specs=_row_blk,
    out_shape=jax.ShapeDtypeStruct((_N, _H), jnp.float32),
)


def _dec_body(s0_r, s1_r, u_r, dv_r, b_r, w1_r, b1_r, w2_r, b2_r, o_r):
    h9 = dv_r[...] * (s0_r[...] + s1_r[...] + u_r[...]) + b_r[...]
    d = jnp.maximum(
        jnp.dot(h9, w1_r[...], preferred_element_type=jnp.float32)
        + b1_r[...], 0.0)
    o_r[...] = jnp.dot(d, w2_r[...], preferred_element_type=jnp.float32) \
        + b2_r[...]


_dec_call = pl.pallas_call(
    _dec_body,
    grid=(_G,),
    in_specs=[_row_blk, _row_blk_hi, _row_blk, _row_blk, _full_b, _full_w,
              _full_b, pl.BlockSpec((_H, 3), lambda i: (0, 0)),
              pl.BlockSpec((1, 3), lambda i: (0, 0))],
    out_specs=pl.BlockSpec((_BLK, 3), lambda i: (i, 0)),
    out_shape=jax.ShapeDtypeStruct((_N, 3), jnp.float32),
)


def kernel(x, edge_index, enc_W1, enc_b1, enc_W2, enc_b2, gcn_W, gcn_b,
           dec_W1, dec_b1, dec_W2, dec_b2):
    src = edge_index[0].reshape(_NW, _NB, _BCH, _CH)
    dst = edge_index[1].reshape(_NW, _NB, _BCH, _CH)
    cnt = _sc_cnt(edge_index[1].reshape(_NW, _NCHUNK, _CH))
    h2 = _enc1_call(x, enc_W1, enc_b1.reshape(1, _H),
                    enc_W2, enc_b2.reshape(1, _H))
    u, dinvB = _enc2_call(h2, cnt, cnt, gcn_W[0])
    L = gcn_W.shape[0]
    for i in range(L):
        S = _sc_agg(u, src, dst)
        b_i = gcn_b[i].reshape(1, _H)
        if i < L - 1:
            u = _mid_call(S, S, u, dinvB, b_i, gcn_W[i + 1])
        else:
            out = _dec_call(S, S, u, dinvB, b_i, dec_W1,
                            dec_b1.reshape(1, _H), dec_W2,
                            dec_b2.reshape(1, 3))
    return out


# speedup vs baseline: 1.7127x; 1.1925x over previous
import jax
import jax.numpy as jnp
from jax import lax
from jax.experimental import pallas as pl
from jax.experimental.pallas import tpu as pltpu
from jax.experimental.pallas import tpu_sc as plsc

_N = 10000
_E = 320000
_H = 128
_NC = 2
_NS = 16
_NW = _NC * _NS
_EPT = _E // _NW
_CH = 80
_NCHUNK = _EPT // _CH
_BCH = 25
_NB = _NCHUNK // _BCH
_RPT = 624
_TAIL = _N - _NS * _RPT
_ZN = _RPT // _CH
_ZREM = _RPT - _ZN * _CH

_mesh = plsc.VectorSubcoreMesh(core_axis_name="c", subcore_axis_name="s")


def _fill(ref, nrows, ncols, value):
    v = jnp.full((16,), value, jnp.float32)
    for r in range(nrows):
        for j in range(ncols // 16):
            ref[r, pl.ds(j * 16, 16)] = v


def _clear_rows(zb, acc, s):
    base = s * _RPT
    for t in range(_ZN):
        pltpu.sync_copy(zb, acc.at[pl.ds(base + t * _CH, _CH)])
    if _ZREM:
        pltpu.sync_copy(zb.at[pl.ds(0, _ZREM)],
                        acc.at[pl.ds(base + _ZN * _CH, _ZREM)])

    @pl.when(s == 0)
    def _():
        pltpu.sync_copy(zb.at[pl.ds(0, _TAIL)],
                        acc.at[pl.ds(_NS * _RPT, _TAIL)])


def _copy_out(acc, out_hbm, c, s):
    row0 = s * _RPT
    pltpu.sync_copy(acc.at[pl.ds(row0, _RPT)],
                    out_hbm.at[pl.ds(c * _N + row0, _RPT)])

    @pl.when(s == 0)
    def _():
        pltpu.sync_copy(acc.at[pl.ds(_NS * _RPT, _TAIL)],
                        out_hbm.at[pl.ds(c * _N + _NS * _RPT, _TAIL)])


def _agg_body(u_hbm, src4_hbm, dst4_hbm, out_hbm, acc, sb0, db0, sb1, db1,
              rows0, rows1, rows2, semi0, semi1, sem0, sem1, sem2):
    c = lax.axis_index("c")
    s = lax.axis_index("s")
    wid = c * _NS + s
    sb = (sb0, sb1)
    db = (db0, db1)
    semi = (semi0, semi1)
    rows = (rows0, rows1, rows2)
    sems = (sem0, sem1, sem2)

    def idx_issue(blk):
        sl = blk % 2
        pltpu.async_copy(src4_hbm.at[wid, blk], sb[sl], semi[sl])
        pltpu.async_copy(dst4_hbm.at[wid, blk], db[sl], semi[sl])

    def idx_wait(sl):
        pltpu.make_async_copy(src4_hbm.at[wid, 0], sb[sl], semi[sl]).wait()
        pltpu.make_async_copy(dst4_hbm.at[wid, 0], db[sl], semi[sl]).wait()

    idx_issue(0)
    idx_issue(1)
    _fill(rows0, _CH, _H, 0.0)
    _clear_rows(rows0, acc, s)
    idx_wait(0)
    plsc.subcore_barrier()

    def issue(sl, q, b):
        pltpu.async_copy(u_hbm.at[sb[sl].at[q]], rows[b], sems[b])

    def consume(sl, q, b):
        pltpu.make_async_copy(u_hbm.at[pl.ds(0, _CH)], rows[b],
                              sems[b]).wait()
        pltpu.sync_copy(rows[b], acc.at[db[sl].at[q]], add=True)

    issue(0, 0, 0)
    issue(0, 1, 1)
    issue(0, 2, 2)

    for blk in range(_NB):
        sl = blk % 2
        nsl = 1 - sl
        ph = blk % 3
        last = blk == _NB - 1

        @pl.loop(0, _BCH - 4, step=3)
        def _(q, sl=sl, ph=ph):
            consume(sl, q, ph)
            issue(sl, q + 3, ph)
            consume(sl, q + 1, (ph + 1) % 3)
            issue(sl, q + 4, (ph + 1) % 3)
            consume(sl, q + 2, (ph + 2) % 3)
            issue(sl, q + 5, (ph + 2) % 3)

        if not last:
            idx_wait(nsl)
        consume(sl, _BCH - 4, ph)
        issue(sl, _BCH - 1, ph)
        consume(sl, _BCH - 3, (ph + 1) % 3)
        if not last:
            issue(nsl, 0, (ph + 1) % 3)
        consume(sl, _BCH - 2, (ph + 2) % 3)
        if not last:
            issue(nsl, 1, (ph + 2) % 3)
        consume(sl, _BCH - 1, ph)
        if not last:
            issue(nsl, 2, ph)
            if blk + 2 < _NB:
                idx_issue(blk + 2)

    plsc.subcore_barrier()
    _copy_out(acc, out_hbm, c, s)


_sc_agg = pl.kernel(
    _agg_body,
    out_type=jax.ShapeDtypeStruct((2 * _N, _H), jnp.float32),
    mesh=_mesh,
    scratch_types=[
        pltpu.VMEM_SHARED((_N, _H), jnp.float32),
        pltpu.VMEM((_BCH, _CH), jnp.int32),
        pltpu.VMEM((_BCH, _CH), jnp.int32),
        pltpu.VMEM((_BCH, _CH), jnp.int32),
        pltpu.VMEM((_BCH, _CH), jnp.int32),
        pltpu.VMEM((_CH, _H), jnp.float32),
        pltpu.VMEM((_CH, _H), jnp.float32),
        pltpu.VMEM((_CH, _H), jnp.float32),
        pltpu.SemaphoreType.DMA,
        pltpu.SemaphoreType.DMA,
        pltpu.SemaphoreType.DMA,
        pltpu.SemaphoreType.DMA,
        pltpu.SemaphoreType.DMA,
    ],
)

_CNTW = 128


def _cnt_body(dst3_hbm, out_hbm, acc, dst_v, ones, zb, semi):
    c = lax.axis_index("c")
    s = lax.axis_index("s")
    wid = c * _NS + s
    p = pltpu.async_copy(dst3_hbm.at[wid], dst_v, semi)
    _fill(zb, _CH, _CNTW, 0.0)
    _clear_rows(zb, acc, s)
    _fill(ones, _CH, _CNTW, 1.0)
    p.wait()
    plsc.subcore_barrier()

    @pl.loop(0, _NCHUNK)
    def _(q):
        pltpu.sync_copy(ones, acc.at[dst_v.at[q]], add=True)

    plsc.subcore_barrier()
    _copy_out(acc, out_hbm, c, s)


_sc_cnt = pl.kernel(
    _cnt_body,
    out_type=jax.ShapeDtypeStruct((2 * _N, _CNTW), jnp.float32),
    mesh=_mesh,
    scratch_types=[
        pltpu.VMEM_SHARED((_N, _CNTW), jnp.float32),
        pltpu.VMEM((_NCHUNK, _CH), jnp.int32),
        pltpu.VMEM((_CH, _CNTW), jnp.float32),
        pltpu.VMEM((_CH, _CNTW), jnp.float32),
        pltpu.SemaphoreType.DMA,
    ],
)

_BLK = 1000
_G = _N // _BLK

_full_w = pl.BlockSpec((_H, _H), lambda i: (0, 0))
_full_b = pl.BlockSpec((1, _H), lambda i: (0, 0))
_row_blk = pl.BlockSpec((_BLK, _H), lambda i: (i, 0))
_row_blk_hi = pl.BlockSpec((_BLK, _H), lambda i: (i + _G, 0))
_cnt_blk = pl.BlockSpec((_BLK, _CNTW), lambda i: (i, 0))
_cnt_blk_hi = pl.BlockSpec((_BLK, _CNTW), lambda i: (i + _G, 0))


def _enc1_body(x_r, w1_r, b1_r, w2_r, b2_r, h_r):
    h = jnp.maximum(
        jnp.dot(x_r[...], w1_r[...], preferred_element_type=jnp.float32)
        + b1_r[...], 0.0)
    h_r[...] = jnp.maximum(
        jnp.dot(h, w2_r[...], preferred_element_type=jnp.float32)
        + b2_r[...], 0.0)


_enc1_call = pl.pallas_call(
    _enc1_body,
    grid=(_G,),
    in_specs=[_row_blk, _full_w, _full_b, _full_w, _full_b],
    out_specs=_row_blk,
    out_shape=jax.ShapeDtypeStruct((_N, _H), jnp.float32),
)


def _enc2_body(h_r, c0_r, c1_r, w0_r, u_r, dv_r):
    deg = c0_r[:, 0:1] + c1_r[:, 0:1] + 1.0
    dinv = lax.rsqrt(deg)
    u_r[...] = dinv * jnp.dot(h_r[...], w0_r[...],
                              preferred_element_type=jnp.float32)
    dv_r[...] = jnp.broadcast_to(dinv, (_BLK, _H))


_enc2_call = pl.pallas_call(
    _enc2_body,
    grid=(_G,),
    in_specs=[_row_blk, _cnt_blk, _cnt_blk_hi, _full_w],
    out_specs=[_row_blk, _row_blk],
    out_shape=[jax.ShapeDtypeStruct((_N, _H), jnp.float32),
               jax.ShapeDtypeStruct((_N, _H), jnp.float32)],
)


def _mid_body(s0_r, s1_r, u_r, dv_r, b_r, w_r, o_r):
    dinv = dv_r[...]
    h = jnp.maximum(dinv * (s0_r[...] + s1_r[...] + u_r[...]) + b_r[...], 0.0)
    o_r[...] = dinv * jnp.dot(h, w_r[...], preferred_element_type=jnp.float32)


_mid_call = pl.pallas_call(
    _mid_body,
    grid=(_G,),
    in_specs=[_row_blk, _row_blk_hi, _row_blk, _row_blk, _full_b, _full_w],
    out_specs=_row_blk,
    out_shape=jax.ShapeDtypeStruct((_N, _H), jnp.float32),
)


def _dec_body(s0_r, s1_r, u_r, dv_r, b_r, w1_r, b1_r, w2_r, b2_r, o_r):
    h9 = dv_r[...] * (s0_r[...] + s1_r[...] + u_r[...]) + b_r[...]
    d = jnp.maximum(
        jnp.dot(h9, w1_r[...], preferred_element_type=jnp.float32)
        + b1_r[...], 0.0)
    o_r[...] = jnp.dot(d, w2_r[...], preferred_element_type=jnp.float32) \
        + b2_r[...]


_dec_call = pl.pallas_call(
    _dec_body,
    grid=(_G,),
    in_specs=[_row_blk, _row_blk_hi, _row_blk, _row_blk, _full_b, _full_w,
              _full_b, pl.BlockSpec((_H, 3), lambda i: (0, 0)),
              pl.BlockSpec((1, 3), lambda i: (0, 0))],
    out_specs=pl.BlockSpec((_BLK, 3), lambda i: (i, 0)),
    out_shape=jax.ShapeDtypeStruct((_N, 3), jnp.float32),
)


def kernel(x, edge_index, enc_W1, enc_b1, enc_W2, enc_b2, gcn_W, gcn_b,
           dec_W1, dec_b1, dec_W2, dec_b2):
    src = edge_index[0].reshape(_NW, _NB, _BCH, _CH)
    dst = edge_index[1].reshape(_NW, _NB, _BCH, _CH)
    cnt = _sc_cnt(edge_index[1].reshape(_NW, _NCHUNK, _CH))
    h2 = _enc1_call(x, enc_W1, enc_b1.reshape(1, _H),
                    enc_W2, enc_b2.reshape(1, _H))
    u, dinvB = _enc2_call(h2, cnt, cnt, gcn_W[0])
    L = gcn_W.shape[0]
    for i in range(L):
        S = _sc_agg(u, src, dst)
        b_i = gcn_b[i].reshape(1, _H)
        if i < L - 1:
            u = _mid_call(S, S, u, dinvB, b_i, gcn_W[i + 1])
        else:
            out = _dec_call(S, S, u, dinvB, b_i, dec_W1,
                            dec_b1.reshape(1, _H), dec_W2,
                            dec_b2.reshape(1, 3))
    return out
